# Initial kernel scaffold; baseline (speedup 1.0000x reference)
import jax
import jax.numpy as jnp
from jax.experimental import pallas as pl


def kernel(tokens, path_table, bin_table, pe, zc_W, zc_b, comb_W, comb_b, null_emb):
    raise NotImplementedError("write your pallas kernel here")


# speedup vs baseline: 4.8812x; 4.8812x over previous
import functools

import jax
import jax.numpy as jnp
from jax import lax
from jax.experimental import pallas as pl
from jax.experimental.pallas import tpu as pltpu
from jax.experimental.pallas import tpu_sc as plsc

B = 16384
D = 128
Z_BINS = 64
T_ROWS = 136

_NC = 2
_NS = 16
NW = _NC * _NS
BPW = B // NW
CHUNK = 128
NCH = BPW // CHUNK


def _table_body(path_ref, bin_ref, pe_ref, zcw_ref, zcb_ref, cw_ref, cb_ref,
                null_ref, out_ref):
    row = lax.broadcasted_iota(jnp.int32, (2 * Z_BINS, 1), 0)
    zb = row % Z_BINS
    zcol = (126 * zb + 63) // Z_BINS
    col = lax.broadcasted_iota(jnp.int32, (2 * Z_BINS, 2 * Z_BINS), 1)
    oh_sin = (col == zcol).astype(jnp.float32)
    sin128 = jnp.dot(oh_sin, pe_ref[...], preferred_element_type=jnp.float32)
    bin128 = jnp.concatenate([bin_ref[...], bin_ref[...]], axis=0)
    z_cat = jnp.concatenate([bin128, sin128], axis=1)
    z_emb = lax.dot_general(z_cat, zcw_ref[...], (((1,), (1,)), ((), ())),
                            preferred_element_type=jnp.float32) + zcb_ref[...]
    pmask = (row < Z_BINS).astype(jnp.float32)
    path128 = pmask * path_ref[0:1, :] + (1.0 - pmask) * path_ref[1:2, :]
    combined = jnp.concatenate([path128, z_emb], axis=1)
    emb = lax.dot_general(combined, cw_ref[...], (((1,), (1,)), ((), ())),
                          preferred_element_type=jnp.float32) + cb_ref[...]
    out_ref[0:2 * Z_BINS, :] = emb
    out_ref[2 * Z_BINS:T_ROWS, :] = jnp.broadcast_to(null_ref[...],
                                                     (T_ROWS - 2 * Z_BINS, D))


def _build_table(path_table, bin_table, pe_p, zc_W, zc_b2, comb_W, comb_b2,
                 null_emb):
    return pl.pallas_call(
        _table_body,
        out_shape=jax.ShapeDtypeStruct((T_ROWS, D), jnp.float32),
    )(path_table, bin_table, pe_p, zc_W, zc_b2, comb_W, comb_b2, null_emb)


def _sc_gather_body(table_hbm, tok_hbm, out_hbm, idx_v, rows_v, sem):
    wid = lax.axis_index("s") * _NC + lax.axis_index("c")
    pltpu.sync_copy(tok_hbm.at[wid], idx_v)
    copies = []
    for j in range(NCH):
        copies.append(
            pltpu.async_copy(table_hbm.at[idx_v.at[j]],
                             rows_v.at[pl.ds(j * CHUNK, CHUNK)], sem))
    for c in copies:
        c.wait()
    pltpu.sync_copy(rows_v, out_hbm.at[wid])


@functools.cache
def _sc_gather():
    mesh = plsc.VectorSubcoreMesh(core_axis_name="c", subcore_axis_name="s")
    return pl.kernel(
        _sc_gather_body,
        mesh=mesh,
        out_type=jax.ShapeDtypeStruct((NW, BPW, D), jnp.float32),
        scratch_types=[
            pltpu.VMEM((NCH, CHUNK), jnp.int32),
            pltpu.VMEM((BPW, D), jnp.float32),
            pltpu.SemaphoreType.DMA,
        ],
    )


def kernel(tokens, path_table, bin_table, pe, zc_W, zc_b, comb_W, comb_b,
           null_emb):
    pe_p = jnp.concatenate([pe, jnp.zeros((1, D), jnp.float32)], axis=0)
    table = _build_table(path_table, bin_table, pe_p, zc_W,
                         zc_b.reshape(1, D), comb_W, comb_b.reshape(1, D),
                         null_emb)
    toks = tokens.astype(jnp.int32).reshape(NW, NCH, CHUNK)
    out = _sc_gather()(table, toks)
    return out.reshape(B, D)


# speedup vs baseline: 6.4992x; 1.3315x over previous
import functools

import jax
import jax.numpy as jnp
from jax import lax
from jax.experimental import pallas as pl
from jax.experimental.pallas import tpu as pltpu
from jax.experimental.pallas import tpu_sc as plsc

B = 16384
D = 128
Z_BINS = 64
T_ROWS = 136

_NC = 2
_NS = 16
NW = _NC * _NS
BPW = B // NW
CHUNK = 128
NCH = BPW // CHUNK
NREP = 32


def _table_body(path_ref, bin_ref, pe_ref, zcw_ref, zcb_ref, cw_ref, cb_ref,
                null_ref, out_ref):
    row = lax.broadcasted_iota(jnp.int32, (2 * Z_BINS, 1), 0)
    zb = row % Z_BINS
    zcol = (126 * zb + 63) // Z_BINS
    col = lax.broadcasted_iota(jnp.int32, (2 * Z_BINS, 2 * Z_BINS), 1)
    oh_sin = (col == zcol).astype(jnp.float32)
    pe_p = jnp.concatenate([pe_ref[...], jnp.zeros((1, D), jnp.float32)],
                           axis=0)
    sin128 = jnp.dot(oh_sin, pe_p, preferred_element_type=jnp.float32)
    bin128 = jnp.concatenate([bin_ref[...], bin_ref[...]], axis=0)
    z_cat = jnp.concatenate([bin128, sin128], axis=1)
    z_emb = lax.dot_general(z_cat, zcw_ref[...], (((1,), (1,)), ((), ())),
                            preferred_element_type=jnp.float32) + zcb_ref[...]
    pmask = (row < Z_BINS).astype(jnp.float32)
    path128 = pmask * path_ref[0:1, :] + (1.0 - pmask) * path_ref[1:2, :]
    combined = jnp.concatenate([path128, z_emb], axis=1)
    emb = lax.dot_general(combined, cw_ref[...], (((1,), (1,)), ((), ())),
                          preferred_element_type=jnp.float32) + cb_ref[...]
    null_rows = jnp.broadcast_to(null_ref[...], (T_ROWS - 2 * Z_BINS, D))
    for w in range(NREP):
        out_ref[pl.ds(w * T_ROWS, 2 * Z_BINS), :] = emb
        out_ref[pl.ds(w * T_ROWS + 2 * Z_BINS, T_ROWS - 2 * Z_BINS), :] = (
            null_rows)


def _build_table(path_table, bin_table, pe_p, zc_W, zc_b2, comb_W, comb_b2,
                 null_emb):
    return pl.pallas_call(
        _table_body,
        out_shape=jax.ShapeDtypeStruct((NREP * T_ROWS, D), jnp.float32),
    )(path_table, bin_table, pe_p, zc_W, zc_b2, comb_W, comb_b2, null_emb)


def _sc_gather_body(table_hbm, tok_hbm, out_hbm, idx_v, rows_v, g_sem, o_sem):
    wid = lax.axis_index("s") * _NC + lax.axis_index("c")
    pltpu.sync_copy(tok_hbm.at[wid], idx_v)
    off = (wid % NREP) * T_ROWS
    for j in range(NCH):
        for k in range(CHUNK // 16):
            sl = (j, pl.ds(k * 16, 16))
            idx_v[sl] = idx_v[sl] + off
    gathers = []
    for j in range(NCH):
        gathers.append(
            pltpu.async_copy(table_hbm.at[idx_v.at[j]],
                             rows_v.at[pl.ds(j * CHUNK, CHUNK)], g_sem))
    outs = []
    for h in range(2):
        gathers[2 * h].wait()
        gathers[2 * h + 1].wait()
        outs.append(
            pltpu.async_copy(
                rows_v.at[pl.ds(h * (BPW // 2), BPW // 2)],
                out_hbm.at[wid, pl.ds(h * (BPW // 2), BPW // 2)], o_sem))
    for c in outs:
        c.wait()


@functools.cache
def _sc_gather():
    mesh = plsc.VectorSubcoreMesh(core_axis_name="c", subcore_axis_name="s")
    return pl.kernel(
        _sc_gather_body,
        mesh=mesh,
        out_type=jax.ShapeDtypeStruct((NW, BPW, D), jnp.float32),
        scratch_types=[
            pltpu.VMEM((NCH, CHUNK), jnp.int32),
            pltpu.VMEM((BPW, D), jnp.float32),
            pltpu.SemaphoreType.DMA,
            pltpu.SemaphoreType.DMA,
        ],
    )


def kernel(tokens, path_table, bin_table, pe, zc_W, zc_b, comb_W, comb_b,
           null_emb):
    table = _build_table(path_table, bin_table, pe, zc_W,
                         zc_b.reshape(1, D), comb_W, comb_b.reshape(1, D),
                         null_emb)
    toks = tokens.astype(jnp.int32).reshape(NW, NCH, CHUNK)
    out = _sc_gather()(table, toks)
    return out.reshape(B, D)
